# baseline (speedup 1.0000x reference)
import functools

import jax
import jax.numpy as jnp
from jax import lax
from jax.experimental import pallas as pl
from jax.experimental.pallas import tpu as pltpu
from jax.experimental.pallas import tpu_sc as plsc

_L = 16
_NC = 2
_NS = 16
_NW = _NC * _NS
_ROWS = 8
_NSLOT = 3
_LUT_N = 1024


def _sc_body(x_hbm, s_hbm, lut_hbm, out_hbm,
             buf0, buf1, buf2, s_v, lut_v,
             semi0, semi1, semi2, semo0, semo1, semo2):
  b_, rows, cols = x_hbm.shape
  rows_per_w = (b_ * rows) // _NW
  nch = rows_per_w // _ROWS
  w_per_b = rows // rows_per_w

  wid = lax.axis_index("c") * _NS + lax.axis_index("s")
  d0 = wid // w_per_b
  row0 = (wid % w_per_b) * rows_per_w

  bufs = (buf0, buf1, buf2)
  sin = (semi0, semi1, semi2)
  sout = (semo0, semo1, semo2)

  pltpu.sync_copy(s_hbm, s_v)
  pltpu.sync_copy(lut_hbm, lut_v)
  s_vec = s_v[...]

  def start_in(g, s):
    pltpu.async_copy(
        x_hbm.at[d0, pl.ds(row0 + g * _ROWS, _ROWS), :], bufs[s], sin[s])

  def wait_in(s):
    pltpu.make_async_copy(
        x_hbm.at[0, pl.ds(0, _ROWS), :], bufs[s], sin[s]).wait()

  def start_out(g, s):
    pltpu.async_copy(
        bufs[s], out_hbm.at[d0, pl.ds(row0 + g * _ROWS, _ROWS), :], sout[s])

  def wait_out(s):
    pltpu.make_async_copy(
        bufs[s], out_hbm.at[0, pl.ds(0, _ROWS), :], sout[s]).wait()

  def compute(s):
    buf = bufs[s]
    for j in range(_ROWS):
      @plsc.parallel_loop(0, cols, step=_L, unroll=8)
      def _(off):
        v = buf[j, pl.ds(off, _L)]
        u = v * s_vec + 256.0
        u = jnp.minimum(u, 512.0)
        u = jnp.maximum(u, 0.0)
        idx = u.astype(jnp.int32)
        buf[j, pl.ds(off, _L)] = plsc.load_gather(lut_v, [idx])

  start_in(0, 0)
  start_in(1, 1)

  wait_in(0)
  compute(0)
  start_out(0, 0)
  start_in(2, 2)

  @pl.loop(1, nch - 3, step=_NSLOT)
  def _steady(g0):
    for d in range(_NSLOT):
      s = (1 + d) % _NSLOT
      sp = d % _NSLOT
      g = g0 + d
      wait_in(s)
      compute(s)
      start_out(g, s)
      wait_out(sp)
      start_in(g + 2, sp)

  for g in (nch - 3, nch - 2, nch - 1):
    s = g % _NSLOT
    sp = (g - 1) % _NSLOT
    wait_in(s)
    compute(s)
    start_out(g, s)
    wait_out(sp)
    if g + 2 < nch:
      start_in(g + 2, sp)

  wait_out((nch - 1) % _NSLOT)


@functools.lru_cache(maxsize=None)
def _build_kernel(shape):
  b_, rows, cols = shape
  rows_per_w = (b_ * rows) // _NW
  assert (b_ * rows) % _NW == 0 and rows_per_w % _ROWS == 0, shape
  nch = rows_per_w // _ROWS
  assert nch % _NSLOT == 1 and nch >= 4, shape
  mesh = plsc.VectorSubcoreMesh(
      core_axis_name="c", subcore_axis_name="s",
      num_cores=_NC, num_subcores=_NS)
  return pl.kernel(
      _sc_body,
      out_type=jax.ShapeDtypeStruct(shape, jnp.float32),
      mesh=mesh,
      compiler_params=pltpu.CompilerParams(needs_layout_passes=False),
      scratch_types=[
          pltpu.VMEM((_ROWS, cols), jnp.float32),
          pltpu.VMEM((_ROWS, cols), jnp.float32),
          pltpu.VMEM((_ROWS, cols), jnp.float32),
          pltpu.VMEM((_L,), jnp.float32),
          pltpu.VMEM((_LUT_N,), jnp.float32),
          pltpu.SemaphoreType.DMA,
          pltpu.SemaphoreType.DMA,
          pltpu.SemaphoreType.DMA,
          pltpu.SemaphoreType.DMA,
          pltpu.SemaphoreType.DMA,
          pltpu.SemaphoreType.DMA,
      ],
  )


def kernel(x, raw_alpha, levels):
  alpha = jax.nn.softplus(raw_alpha)

  t_rep = (jnp.arange(513, dtype=jnp.float32) - 255.5) * (1.0 / 256.0)
  dist = jnp.abs(t_rep[:, None] - levels[None, :])
  lut = levels[jnp.argmin(dist, axis=1)] * alpha
  lut_pad = jnp.zeros((_LUT_N,), jnp.float32).at[:513].set(lut)
  s_arr = jnp.full((_L,), 256.0 / alpha, dtype=jnp.float32)

  return _build_kernel(x.shape)(x, s_arr, lut_pad)


# speedup vs baseline: 1.1238x; 1.1238x over previous
import functools

import jax
import jax.numpy as jnp
from jax import lax
from jax.experimental import pallas as pl
from jax.experimental.pallas import tpu as pltpu
from jax.experimental.pallas import tpu_sc as plsc

_L = 16
_NC = 2
_NS = 16
_NW = _NC * _NS
_ROWS = 4
_NBUF = 2
_LUT_N = 1024

_RS_SC = 8192
_BR = 256


def _sc_body(rs, x_hbm, s_hbm, lut_hbm, out_hbm,
             in0, in1, out0, out1, s_v, lut_v,
             sem_in0, sem_in1, sem_out0, sem_out1):
  b_, rows, cols = x_hbm.shape
  rpw = rs // _NW
  nch = rpw // _ROWS

  wid = lax.axis_index("c") * _NS + lax.axis_index("s")
  fr = wid * rpw
  d0 = fr // rows
  row0 = fr % rows

  ins = (in0, in1)
  outs = (out0, out1)
  sin = (sem_in0, sem_in1)
  sout = (sem_out0, sem_out1)

  pltpu.sync_copy(s_hbm, s_v)
  pltpu.sync_copy(lut_hbm, lut_v)
  s_vec = s_v[...]

  def start_in(g, b):
    pltpu.async_copy(
        x_hbm.at[d0, pl.ds(row0 + g * _ROWS, _ROWS), :], ins[b], sin[b])

  def wait_in(b):
    pltpu.make_async_copy(
        x_hbm.at[0, pl.ds(0, _ROWS), :], ins[b], sin[b]).wait()

  def start_out(g, b):
    pltpu.async_copy(
        outs[b], out_hbm.at[d0, pl.ds(row0 + g * _ROWS, _ROWS), :], sout[b])

  def wait_out(b):
    pltpu.make_async_copy(
        outs[b], out_hbm.at[0, pl.ds(0, _ROWS), :], sout[b]).wait()

  def compute(b):
    src = ins[b]
    dst = outs[b]
    for j in range(_ROWS):
      @plsc.parallel_loop(0, cols, step=_L, unroll=8)
      def _(off):
        v = src[j, pl.ds(off, _L)]
        u = v * s_vec + 256.0
        u = jnp.minimum(u, 512.0)
        u = jnp.maximum(u, 0.0)
        idx = u.astype(jnp.int32)
        dst[j, pl.ds(off, _L)] = plsc.load_gather(lut_v, [idx])

  for b in range(_NBUF):
    start_in(b, b)

  for g in range(_NBUF):
    b = g
    wait_in(b)
    compute(b)
    start_out(g, b)
    start_in(g + _NBUF, b)

  @pl.loop(_NBUF, nch - _NBUF, step=_NBUF)
  def _steady(g0):
    for b in range(_NBUF):
      g = g0 + b
      wait_in(b)
      wait_out(b)
      compute(b)
      start_out(g, b)
      start_in(g + _NBUF, b)

  for g in range(nch - _NBUF, nch):
    b = g % _NBUF
    wait_in(b)
    wait_out(b)
    compute(b)
    start_out(g, b)

  for b in range(_NBUF):
    wait_out(b)


def _tc_gather128(tab, idx):
  dnums = lax.GatherDimensionNumbers(
      offset_dims=(), collapsed_slice_dims=(2,), start_index_map=(2,),
      operand_batching_dims=(0, 1), start_indices_batching_dims=(0, 1))
  return lax.gather(
      tab, idx[..., None], dnums, slice_sizes=(1, 1, 1),
      mode=lax.GatherScatterMode.PROMISE_IN_BOUNDS)


def _tc_body(s_ref, tab_ref, x_ref, scout_ref, o_ref):
  del scout_ref
  s = s_ref[0, 0]
  x = x_ref[...]
  u = jnp.minimum(jnp.abs(x) * s, 255.0)
  idx = u.astype(jnp.int32)
  low = idx & 127
  tab0 = jnp.broadcast_to(tab_ref[:, :, :128], (1, _BR, 128))
  tab1 = jnp.broadcast_to(tab_ref[:, :, 128:], (1, _BR, 128))
  v = jnp.where(idx >= 128,
                _tc_gather128(tab1, low),
                _tc_gather128(tab0, low))
  o_ref[...] = jnp.where(x < 0.0, -v, v)


@functools.lru_cache(maxsize=None)
def _build_sc_kernel(shape, rs):
  b_, rows, cols = shape
  rpw = rs // _NW
  assert rs % (_NW * _ROWS) == 0 and (rpw // _ROWS) >= 2 * _NBUF, (shape, rs)
  assert rs <= rows or rows % rpw == 0, (shape, rs)
  mesh = plsc.VectorSubcoreMesh(
      core_axis_name="c", subcore_axis_name="s",
      num_cores=_NC, num_subcores=_NS)
  return pl.kernel(
      functools.partial(_sc_body, rs),
      out_type=jax.ShapeDtypeStruct(shape, jnp.float32),
      mesh=mesh,
      compiler_params=pltpu.CompilerParams(needs_layout_passes=False),
      scratch_types=[
          pltpu.VMEM((_ROWS, cols), jnp.float32),
          pltpu.VMEM((_ROWS, cols), jnp.float32),
          pltpu.VMEM((_ROWS, cols), jnp.float32),
          pltpu.VMEM((_ROWS, cols), jnp.float32),
          pltpu.VMEM((_L,), jnp.float32),
          pltpu.VMEM((_LUT_N,), jnp.float32),
          pltpu.SemaphoreType.DMA,
          pltpu.SemaphoreType.DMA,
          pltpu.SemaphoreType.DMA,
          pltpu.SemaphoreType.DMA,
      ],
  )


@functools.lru_cache(maxsize=None)
def _build_tc_kernel(shape, rs):
  b_, rows, cols = shape
  n_tc = b_ * rows - rs
  assert n_tc % _BR == 0 and rows % _BR == 0 and rs % _BR == 0, (shape, rs)

  def _map(t):
    fr = rs + t * _BR
    return (fr // rows, (fr % rows) // _BR, 0)

  return pl.pallas_call(
      _tc_body,
      grid=(n_tc // _BR,),
      in_specs=[
          pl.BlockSpec(memory_space=pltpu.MemorySpace.SMEM),
          pl.BlockSpec((1, 1, 256), lambda t: (0, 0, 0)),
          pl.BlockSpec((1, _BR, cols), _map),
          pl.BlockSpec(memory_space=pltpu.MemorySpace.HBM),
      ],
      out_specs=pl.BlockSpec((1, _BR, cols), _map),
      out_shape=jax.ShapeDtypeStruct(shape, jnp.float32),
      input_output_aliases={3: 0},
  )


def kernel(x, raw_alpha, levels):
  alpha = jax.nn.softplus(raw_alpha)

  t_rep = (jnp.arange(513, dtype=jnp.float32) - 255.5) * (1.0 / 256.0)
  dist = jnp.abs(t_rep[:, None] - levels[None, :])
  lut = levels[jnp.argmin(dist, axis=1)] * alpha
  lut_pad = jnp.zeros((_LUT_N,), jnp.float32).at[:513].set(lut)
  s_arr = jnp.full((_L,), 256.0 / alpha, dtype=jnp.float32)
  s_smem = jnp.full((1, 1), 256.0 / alpha, dtype=jnp.float32)
  tp_rep = (jnp.arange(256, dtype=jnp.float32) + 0.5) * (1.0 / 256.0)
  dist_p = jnp.abs(tp_rep[:, None] - levels[None, :])
  plut = levels[jnp.argmin(dist_p, axis=1)] * alpha
  tab3 = plut.reshape(1, 1, 256)

  sc_out = _build_sc_kernel(x.shape, _RS_SC)(x, s_arr, lut_pad)
  return _build_tc_kernel(x.shape, _RS_SC)(s_smem, tab3, x, sc_out)


# speedup vs baseline: 1.4842x; 1.3207x over previous
import functools

import jax
import jax.numpy as jnp
from jax import lax
from jax.experimental import pallas as pl
from jax.experimental.pallas import tpu as pltpu
from jax.experimental.pallas import tpu_sc as plsc

_L = 16
_NC = 2
_NS = 16
_NW = _NC * _NS
_ROWS = 4
_NBUF = 2
_LUT_N = 1024

_RS_SC = 6144
_BR = 256


def _sc_body(rs, x_hbm, s_hbm, lut_hbm, out_hbm,
             in0, in1, out0, out1, s_v, lut_v,
             sem_in0, sem_in1, sem_out0, sem_out1):
  b_, rows, cols = x_hbm.shape
  rpw = rs // _NW
  nch = rpw // _ROWS

  wid = lax.axis_index("c") * _NS + lax.axis_index("s")
  fr = wid * rpw
  d0 = fr // rows
  row0 = fr % rows

  ins = (in0, in1)
  outs = (out0, out1)
  sin = (sem_in0, sem_in1)
  sout = (sem_out0, sem_out1)

  pltpu.sync_copy(s_hbm, s_v)
  pltpu.sync_copy(lut_hbm, lut_v)
  s_vec = s_v[...]

  def start_in(g, b):
    pltpu.async_copy(
        x_hbm.at[d0, pl.ds(row0 + g * _ROWS, _ROWS), :], ins[b], sin[b])

  def wait_in(b):
    pltpu.make_async_copy(
        x_hbm.at[0, pl.ds(0, _ROWS), :], ins[b], sin[b]).wait()

  def start_out(g, b):
    pltpu.async_copy(
        outs[b], out_hbm.at[d0, pl.ds(row0 + g * _ROWS, _ROWS), :], sout[b])

  def wait_out(b):
    pltpu.make_async_copy(
        outs[b], out_hbm.at[0, pl.ds(0, _ROWS), :], sout[b]).wait()

  def compute(b):
    src = ins[b]
    dst = outs[b]
    for j in range(_ROWS):
      @plsc.parallel_loop(0, cols, step=_L, unroll=8)
      def _(off):
        v = src[j, pl.ds(off, _L)]
        u = v * s_vec + 256.0
        u = jnp.minimum(u, 512.0)
        u = jnp.maximum(u, 0.0)
        idx = u.astype(jnp.int32)
        dst[j, pl.ds(off, _L)] = plsc.load_gather(lut_v, [idx])

  for b in range(_NBUF):
    start_in(b, b)

  for g in range(_NBUF):
    b = g
    wait_in(b)
    compute(b)
    start_out(g, b)
    start_in(g + _NBUF, b)

  @pl.loop(_NBUF, nch - _NBUF, step=_NBUF)
  def _steady(g0):
    for b in range(_NBUF):
      g = g0 + b
      wait_in(b)
      wait_out(b)
      compute(b)
      start_out(g, b)
      start_in(g + _NBUF, b)

  for g in range(nch - _NBUF, nch):
    b = g % _NBUF
    wait_in(b)
    wait_out(b)
    compute(b)
    start_out(g, b)

  for b in range(_NBUF):
    wait_out(b)


def _tc_gather128(tab, idx):
  dnums = lax.GatherDimensionNumbers(
      offset_dims=(), collapsed_slice_dims=(2,), start_index_map=(2,),
      operand_batching_dims=(0, 1), start_indices_batching_dims=(0, 1))
  return lax.gather(
      tab, idx[..., None], dnums, slice_sizes=(1, 1, 1),
      mode=lax.GatherScatterMode.PROMISE_IN_BOUNDS)


def _tc_body(s_ref, tab_ref, x_ref, scout_ref, o_ref):
  del scout_ref
  s = s_ref[0, 0]
  alpha = s_ref[0, 1]
  x = x_ref[...]
  u = jnp.minimum(jnp.abs(x) * s, 255.0)
  idx = u.astype(jnp.int32)
  low = idx & 127
  tab = jnp.broadcast_to(tab_ref[...], (1, _BR, 128))
  g = _tc_gather128(tab, low)
  gi = lax.bitcast_convert_type(g, jnp.int32)
  w = jnp.where(idx >= 128, gi & jnp.int32(-65536), gi << 16)
  xbits = lax.bitcast_convert_type(x, jnp.int32)
  sbit = xbits & jnp.int32(-2147483648)
  val = lax.bitcast_convert_type(w | sbit, jnp.float32)
  o_ref[...] = val * alpha


@functools.lru_cache(maxsize=None)
def _build_sc_kernel(shape, rs):
  b_, rows, cols = shape
  rpw = rs // _NW
  assert rs % (_NW * _ROWS) == 0 and (rpw // _ROWS) >= 2 * _NBUF, (shape, rs)
  assert rs <= rows or rows % rpw == 0, (shape, rs)
  mesh = plsc.VectorSubcoreMesh(
      core_axis_name="c", subcore_axis_name="s",
      num_cores=_NC, num_subcores=_NS)
  return pl.kernel(
      functools.partial(_sc_body, rs),
      out_type=jax.ShapeDtypeStruct(shape, jnp.float32),
      mesh=mesh,
      compiler_params=pltpu.CompilerParams(needs_layout_passes=False),
      scratch_types=[
          pltpu.VMEM((_ROWS, cols), jnp.float32),
          pltpu.VMEM((_ROWS, cols), jnp.float32),
          pltpu.VMEM((_ROWS, cols), jnp.float32),
          pltpu.VMEM((_ROWS, cols), jnp.float32),
          pltpu.VMEM((_L,), jnp.float32),
          pltpu.VMEM((_LUT_N,), jnp.float32),
          pltpu.SemaphoreType.DMA,
          pltpu.SemaphoreType.DMA,
          pltpu.SemaphoreType.DMA,
          pltpu.SemaphoreType.DMA,
      ],
  )


@functools.lru_cache(maxsize=None)
def _build_tc_kernel(shape, rs):
  b_, rows, cols = shape
  n_tc = b_ * rows - rs
  assert n_tc % _BR == 0 and rows % _BR == 0 and rs % _BR == 0, (shape, rs)

  def _map(t):
    fr = rs + t * _BR
    return (fr // rows, (fr % rows) // _BR, 0)

  return pl.pallas_call(
      _tc_body,
      grid=(n_tc // _BR,),
      in_specs=[
          pl.BlockSpec(memory_space=pltpu.MemorySpace.SMEM),
          pl.BlockSpec((1, 1, 128), lambda t: (0, 0, 0)),
          pl.BlockSpec((1, _BR, cols), _map),
          pl.BlockSpec(memory_space=pltpu.MemorySpace.HBM),
      ],
      out_specs=pl.BlockSpec((1, _BR, cols), _map),
      out_shape=jax.ShapeDtypeStruct(shape, jnp.float32),
      input_output_aliases={3: 0},
  )


def kernel(x, raw_alpha, levels):
  alpha = jax.nn.softplus(raw_alpha)

  t_rep = (jnp.arange(513, dtype=jnp.float32) - 255.5) * (1.0 / 256.0)
  dist = jnp.abs(t_rep[:, None] - levels[None, :])
  lut = levels[jnp.argmin(dist, axis=1)] * alpha
  lut_pad = jnp.zeros((_LUT_N,), jnp.float32).at[:513].set(lut)
  s_arr = jnp.full((_L,), 256.0 / alpha, dtype=jnp.float32)
  s_smem = jnp.stack([256.0 / alpha, alpha]).astype(jnp.float32).reshape(1, 2)
  tp_rep = (jnp.arange(256, dtype=jnp.float32) + 0.5) * (1.0 / 256.0)
  dist_p = jnp.abs(tp_rep[:, None] - levels[None, :])
  plut = levels[jnp.argmin(dist_p, axis=1)]
  p16 = lax.bitcast_convert_type(
      plut.astype(jnp.bfloat16), jnp.uint16).astype(jnp.uint32)
  packed = lax.bitcast_convert_type(
      (p16[128:] << 16) | p16[:128], jnp.float32)
  tab3 = packed.reshape(1, 1, 128)

  sc_out = _build_sc_kernel(x.shape, _RS_SC)(x, s_arr, lut_pad)
  return _build_tc_kernel(x.shape, _RS_SC)(s_smem, tab3, x, sc_out)
